# baseline (speedup 1.0000x reference)
import functools

import jax
import jax.numpy as jnp
from jax import lax
from jax.experimental import pallas as pl
from jax.experimental.pallas import tpu as pltpu
from jax.experimental.pallas import tpu_sc as plsc

N = 10000
E = 320000
D = 128
NC, NS, L = 2, 16, 16
NW = NC * NS
EPW = E // NW
C = 80
NCH = EPW // C
GPC = C // L
NVP = 10240
VSL = NVP // NS
RB = 10
RBS = N // RB
EB = 160
EBS = E // EB

_mesh = plsc.VectorSubcoreMesh(
    core_axis_name="c", subcore_axis_name="s", num_cores=NC, num_subcores=NS)
_sc_params = pltpu.CompilerParams(needs_layout_passes=False)


def _dense_body(v_ref, p_ref, wv_ref, bv_ref, wp_ref, bp_ref, w1_ref, b1_ref,
                w2_ref, b2_ref, g1_ref, av_ref, ap_ref, ms_ref, mh_ref):
    v = v_ref[...]
    p = p_ref[...]
    dn = (((1,), (1,)), ((), ()))
    av_ref[...] = lax.dot_general(v, wv_ref[...], dn,
                                  preferred_element_type=jnp.float32) + bv_ref[...]
    ap_ref[...] = lax.dot_general(p, wp_ref[...], dn,
                                  preferred_element_type=jnp.float32) + bp_ref[...]
    h = lax.dot_general(v, w1_ref[...], dn,
                        preferred_element_type=jnp.float32) + b1_ref[...]
    logits = lax.dot_general(h, w2_ref[...], dn,
                             preferred_element_type=jnp.float32) + b2_ref[...]
    z = logits - jnp.log(-jnp.log(g1_ref[...]))
    z0 = z[:, 0:1]
    z1 = z[:, 1:2]
    m = jnp.maximum(z0, z1)
    e0 = jnp.exp(z0 - m)
    e1 = jnp.exp(z1 - m)
    ms_ref[...] = e0 / (e0 + e1)
    mh_ref[...] = (z0 >= z1).astype(jnp.float32)


def _dense(vf, pgf, wv, bv, wp, bp, w1, b1, w2, b2, g1):
    row = lambda i: (i, 0)
    whole = lambda i: (0, 0)
    return pl.pallas_call(
        _dense_body,
        grid=(RB,),
        in_specs=[
            pl.BlockSpec((RBS, D), row),
            pl.BlockSpec((RBS, D), row),
            pl.BlockSpec((D, D), whole),
            pl.BlockSpec((1, D), whole),
            pl.BlockSpec((D, D), whole),
            pl.BlockSpec((1, D), whole),
            pl.BlockSpec((D // 2, D), whole),
            pl.BlockSpec((1, D // 2), whole),
            pl.BlockSpec((2, D // 2), whole),
            pl.BlockSpec((1, 2), whole),
            pl.BlockSpec((RBS, 2), row),
        ],
        out_specs=[
            pl.BlockSpec((RBS, D), row),
            pl.BlockSpec((RBS, D), row),
            pl.BlockSpec((RBS, 1), row),
            pl.BlockSpec((RBS, 1), row),
        ],
        out_shape=[
            jax.ShapeDtypeStruct((N, D), jnp.float32),
            jax.ShapeDtypeStruct((N, D), jnp.float32),
            jax.ShapeDtypeStruct((N, 1), jnp.float32),
            jax.ShapeDtypeStruct((N, 1), jnp.float32),
        ],
    )(vf, pgf, wv, bv, wp, bp, w1, b1, w2, b2, g1)


@functools.partial(
    pl.kernel,
    out_type=jax.ShapeDtypeStruct((E, D), jnp.float32),
    mesh=_mesh,
    compiler_params=_sc_params,
    scratch_types=[
        pltpu.VMEM((2, 2 * 128), jnp.int32),
        pltpu.VMEM((2 * C, D), jnp.float32),
        pltpu.VMEM((2 * C, D), jnp.float32),
        pltpu.VMEM((2 * C, D), jnp.float32),
        pltpu.SemaphoreType.DMA((2,)),
        pltpu.SemaphoreType.DMA((2,)),
        pltpu.SemaphoreType.DMA((2,)),
        pltpu.SemaphoreType.DMA,
    ],
)
def _gsum(av_hbm, ap_hbm, pk_hbm, s_hbm,
          pk, avb, apb, sb, sem_av, sem_ap, sem_out, sem_idx):
    c = lax.axis_index("c")
    s = lax.axis_index("s")
    wid = c * NS + s
    pkb = wid * (NCH * 256)

    def issue_gathers(slot):
        pltpu.async_copy(av_hbm.at[pk.at[slot, pl.ds(0, C)]],
                         avb.at[pl.ds(slot * C, C)], sem_av.at[slot])
        pltpu.async_copy(ap_hbm.at[pk.at[slot, pl.ds(128, C)]],
                         apb.at[pl.ds(slot * C, C)], sem_ap.at[slot])

    def wait_gathers(slot):
        pltpu.make_async_copy(av_hbm.at[pk.at[slot, pl.ds(0, C)]],
                              avb.at[pl.ds(slot * C, C)],
                              sem_av.at[slot]).wait()
        pltpu.make_async_copy(ap_hbm.at[pk.at[slot, pl.ds(128, C)]],
                              apb.at[pl.ds(slot * C, C)],
                              sem_ap.at[slot]).wait()

    def compute_chunk(b, ch, wait_prev_out):
        base = wid * EPW + ch * C

        def row_block(g, _):
            off = g * L
            for e in range(L):
                row = b * C + off + e
                for j in range(D // L):
                    sl = pl.ds(j * L, L)
                    sb[row, sl] = avb[row, sl] + apb[row, sl]
            return 0
        lax.fori_loop(0, GPC, row_block, 0)
        if wait_prev_out:
            pltpu.make_async_copy(
                sb.at[pl.ds(b * C, C)],
                s_hbm.at[pl.ds(wid * EPW + (ch - 2) * C, C)],
                sem_out.at[b]).wait()
        pltpu.async_copy(sb.at[pl.ds(b * C, C)], s_hbm.at[pl.ds(base, C)],
                         sem_out.at[b])

    def iter_body(b, ch, wait_prev_out):
        wait_gathers(b)
        issue_gathers(1 - b)
        d = pltpu.async_copy(pk_hbm.at[pl.ds(pkb + (ch + 2) * 256, 256)],
                             pk.at[b], sem_idx)
        compute_chunk(b, ch, wait_prev_out)
        d.wait()

    pltpu.sync_copy(pk_hbm.at[pl.ds(pkb, 256)], pk.at[0])
    issue_gathers(0)
    pltpu.sync_copy(pk_hbm.at[pl.ds(pkb + 256, 256)], pk.at[1])

    iter_body(0, 0, False)
    iter_body(1, 1, False)

    def chunk_loop(k, _):
        iter_body(0, 2 * k + 2, True)
        iter_body(1, 2 * k + 3, True)
        return 0
    lax.fori_loop(0, (NCH - 5) // 2, chunk_loop, 0)

    iter_body(0, NCH - 3, True)
    wait_gathers(1)
    issue_gathers(0)
    compute_chunk(1, NCH - 2, True)
    wait_gathers(0)
    compute_chunk(0, NCH - 1, True)
    pltpu.make_async_copy(sb.at[pl.ds(C, C)],
                          s_hbm.at[pl.ds(wid * EPW + (NCH - 2) * C, C)],
                          sem_out.at[1]).wait()
    pltpu.make_async_copy(sb.at[pl.ds(0, C)],
                          s_hbm.at[pl.ds(wid * EPW + (NCH - 1) * C, C)],
                          sem_out.at[0]).wait()


def _att_body(s_ref, th_ref, g2_ref, z_ref, ex_ref):
    t = jnp.tanh(s_ref[...])
    att = jnp.sum(t * th_ref[...], axis=1, keepdims=True)
    z = att + g2_ref[...]
    z_ref[...] = z
    ex_ref[...] = jnp.exp(z)


def _att(s, theta, g2):
    row = lambda i: (i, 0)
    return pl.pallas_call(
        _att_body,
        grid=(EB,),
        in_specs=[
            pl.BlockSpec((EBS, D), row),
            pl.BlockSpec((1, D), lambda i: (0, 0)),
            pl.BlockSpec((EBS, 1), row),
        ],
        out_specs=[
            pl.BlockSpec((EBS, 1), row),
            pl.BlockSpec((EBS, 1), row),
        ],
        out_shape=[
            jax.ShapeDtypeStruct((E, 1), jnp.float32),
            jax.ShapeDtypeStruct((E, 1), jnp.float32),
        ],
    )(s, theta, g2)


@functools.partial(
    pl.kernel,
    out_type=[
        jax.ShapeDtypeStruct((NC, NVP), jnp.float32),
        jax.ShapeDtypeStruct((NC, NVP), jnp.float32),
    ],
    mesh=_mesh,
    compiler_params=_sc_params,
    scratch_types=[
        pltpu.VMEM((2, 4 * 128), jnp.int32),
        pltpu.VMEM((C,), jnp.int32),
        pltpu.VMEM((C,), jnp.float32),
        pltpu.VMEM((C,), jnp.float32),
        pltpu.VMEM((NVP,), jnp.float32),
        pltpu.VMEM((NS, VSL), jnp.float32),
        pltpu.VMEM((VSL,), jnp.float32),
        pltpu.VMEM_SHARED((NVP,), jnp.float32),
        pltpu.VMEM_SHARED((NS, NVP), jnp.float32),
        pltpu.SemaphoreType.DMA,
    ],
)
def _stats(pk_hbm, den_hbm, mx_hbm,
           pk, vi_s, zc_v, exc_v, mx_tbl, red_v, slice_v, den_sh, mx_sh,
           sem_idx):
    c = lax.axis_index("c")
    s = lax.axis_index("s")
    wid = c * NS + s
    pkb = wid * (NCH * 512)

    neg = jnp.full((L,), -1e30, jnp.float32)

    def fill_mx(i, _):
        mx_tbl[pl.ds(i * L, L)] = neg
        return 0
    lax.fori_loop(0, NVP // L, fill_mx, 0)

    zv = jnp.zeros((L,), jnp.float32)

    def fill_z(i, _):
        slice_v[pl.ds(i * L, L)] = zv
        return 0
    lax.fori_loop(0, VSL // L, fill_z, 0)
    pltpu.sync_copy(slice_v, den_sh.at[pl.ds(s * VSL, VSL)])
    plsc.subcore_barrier()

    def compute_chunk(b, ch):
        for j in range(C // L):
            sl = pl.ds(j * L, L)
            vi_s[sl] = pk[b, pl.ds(j * L, L)]
            zc_v[sl] = plsc.bitcast(pk[b, pl.ds(256 + j * L, L)], jnp.float32)
            exc_v[sl] = plsc.bitcast(pk[b, pl.ds(384 + j * L, L)], jnp.float32)

        def group_body(g, _):
            off = g * L
            z16 = zc_v[pl.ds(off, L)]
            vi16 = vi_s[pl.ds(off, L)]

            def mx_step(pending):
                cur = plsc.load_gather(mx_tbl, [vi16])
                need = jnp.logical_and(pending, z16 > cur)
                plsc.store_scatter(mx_tbl, [vi16], z16, mask=need)
                cur2 = plsc.load_gather(mx_tbl, [vi16])
                return jnp.logical_and(need, z16 > cur2)
            lax.while_loop(lambda p: jnp.any(p), mx_step,
                           jnp.ones((L,), jnp.bool_))
            return 0
        lax.fori_loop(0, GPC, group_body, 0)
        pltpu.sync_copy(exc_v, den_sh.at[vi_s], add=True)

    def iter_body(b, ch):
        d = pltpu.async_copy(pk_hbm.at[pl.ds(pkb + (ch + 1) * 512, 512)],
                             pk.at[1 - b], sem_idx)
        compute_chunk(b, ch)
        d.wait()

    pltpu.sync_copy(pk_hbm.at[pl.ds(pkb, 512)], pk.at[0])

    def chunk_loop(k, _):
        iter_body(0, 2 * k)
        iter_body(1, 2 * k + 1)
        return 0
    lax.fori_loop(0, (NCH - 1) // 2, chunk_loop, 0)
    compute_chunk(0, NCH - 1)

    plsc.subcore_barrier()
    pltpu.sync_copy(mx_tbl, mx_sh.at[s])
    plsc.subcore_barrier()
    for j in range(NS):
        pltpu.sync_copy(mx_sh.at[j, pl.ds(s * VSL, VSL)], red_v.at[j])

    def red_max(k, _):
        m = red_v[0, pl.ds(k * L, L)]
        for j in range(1, NS):
            m = jnp.maximum(m, red_v[j, pl.ds(k * L, L)])
        slice_v[pl.ds(k * L, L)] = m
        return 0
    lax.fori_loop(0, VSL // L, red_max, 0)
    pltpu.sync_copy(slice_v, mx_hbm.at[c, pl.ds(s * VSL, VSL)])

    pltpu.sync_copy(den_sh.at[pl.ds(s * VSL, VSL)], slice_v)
    pltpu.sync_copy(slice_v, den_hbm.at[c, pl.ds(s * VSL, VSL)])


def _tables_body(denp_ref, mxp_ref, den_ref, mx_ref):
    den_ref[...] = denp_ref[0:1, :] + denp_ref[1:2, :]
    mx_ref[...] = jnp.maximum(mxp_ref[0:1, :], mxp_ref[1:2, :])


def _tables(den_p, mx_p):
    whole = lambda: (0, 0)
    return pl.pallas_call(
        _tables_body,
        grid=(),
        in_specs=[pl.BlockSpec((NC, NVP), whole),
                  pl.BlockSpec((NC, NVP), whole)],
        out_specs=[pl.BlockSpec((1, NVP), whole),
                   pl.BlockSpec((1, NVP), whole)],
        out_shape=[jax.ShapeDtypeStruct((1, NVP), jnp.float32),
                   jax.ShapeDtypeStruct((1, NVP), jnp.float32)],
    )(den_p, mx_p)


@functools.partial(
    pl.kernel,
    out_type=[
        jax.ShapeDtypeStruct((E,), jnp.float32),
        jax.ShapeDtypeStruct((E,), jnp.float32),
        jax.ShapeDtypeStruct((NC, NVP, D), jnp.float32),
    ],
    mesh=_mesh,
    compiler_params=_sc_params,
    scratch_types=[
        pltpu.VMEM((2, 4 * 128), jnp.int32),
        pltpu.VMEM((2 * C, D), jnp.float32),
        pltpu.VMEM((C,), jnp.int32),
        pltpu.VMEM((C,), jnp.float32),
        pltpu.VMEM((C,), jnp.float32),
        pltpu.VMEM((C,), jnp.float32),
        pltpu.VMEM((C,), jnp.float32),
        pltpu.VMEM((NVP,), jnp.float32),
        pltpu.VMEM((NVP,), jnp.float32),
        pltpu.VMEM_SHARED((NVP, D), jnp.float32),
        pltpu.SemaphoreType.DMA((2,)),
        pltpu.SemaphoreType.DMA,
    ],
)
def _edge2(pgf_hbm, pk_hbm, den_hbm, mx_hbm,
           soft_hbm, hard_hbm, agg_hbm,
           pk, pfb, vi_s, zc_v, exc_v, softb, hardb, mx_tbl, den_tbl,
           agg_sh, sem_pf, sem_idx):
    c = lax.axis_index("c")
    s = lax.axis_index("s")
    wid = c * NS + s
    pkb = wid * (NCH * 512)

    pltpu.sync_copy(mx_hbm, mx_tbl)
    pltpu.sync_copy(den_hbm, den_tbl)

    zv = jnp.zeros((L,), jnp.float32)
    for i in range(L):
        for j in range(D // L):
            pfb[i, pl.ds(j * L, L)] = zv
    for r in range(VSL // L):
        pltpu.sync_copy(pfb.at[pl.ds(0, L)],
                        agg_sh.at[pl.ds(s * VSL + r * L, L)])
    plsc.subcore_barrier()

    def issue_gather(slot):
        pltpu.async_copy(pgf_hbm.at[pk.at[slot, pl.ds(128, C)]],
                         pfb.at[pl.ds(slot * C, C)], sem_pf.at[slot])

    def wait_gather(slot):
        pltpu.make_async_copy(pgf_hbm.at[pk.at[slot, pl.ds(128, C)]],
                              pfb.at[pl.ds(slot * C, C)],
                              sem_pf.at[slot]).wait()

    def compute_chunk(b, ch):
        base = wid * EPW + ch * C
        for j in range(C // L):
            sl = pl.ds(j * L, L)
            vi_s[sl] = pk[b, pl.ds(j * L, L)]
            zc_v[sl] = plsc.bitcast(pk[b, pl.ds(256 + j * L, L)], jnp.float32)
            exc_v[sl] = plsc.bitcast(pk[b, pl.ds(384 + j * L, L)], jnp.float32)

        def group_body(g, _):
            off = g * L
            z16 = zc_v[pl.ds(off, L)]
            vi16 = vi_s[pl.ds(off, L)]
            d16 = plsc.load_gather(den_tbl, [vi16])
            m16 = plsc.load_gather(mx_tbl, [vi16])
            soft16 = exc_v[pl.ds(off, L)] / d16
            softb[pl.ds(off, L)] = soft16
            hardb[pl.ds(off, L)] = jnp.where(z16 >= m16, 1.0, 0.0)
            for e in range(L):
                row = off + e
                sc = soft16[e]
                for j in range(D // L):
                    sl = pl.ds(j * L, L)
                    pfb[b * C + row, sl] = pfb[b * C + row, sl] * sc
            return 0
        lax.fori_loop(0, GPC, group_body, 0)

        pltpu.sync_copy(pfb.at[pl.ds(b * C, C)], agg_sh.at[vi_s], add=True)
        pltpu.sync_copy(softb, soft_hbm.at[pl.ds(base, C)])
        pltpu.sync_copy(hardb, hard_hbm.at[pl.ds(base, C)])

    pltpu.sync_copy(pk_hbm.at[pl.ds(pkb, 512)], pk.at[0])
    issue_gather(0)
    pltpu.sync_copy(pk_hbm.at[pl.ds(pkb + 512, 512)], pk.at[1])

    def iter_body(b, ch):
        wait_gather(b)
        issue_gather(1 - b)
        d = pltpu.async_copy(pk_hbm.at[pl.ds(pkb + (ch + 2) * 512, 512)],
                             pk.at[b], sem_idx)
        compute_chunk(b, ch)
        d.wait()

    def chunk_loop(k, _):
        iter_body(0, 2 * k)
        iter_body(1, 2 * k + 1)
        return 0
    lax.fori_loop(0, (NCH - 3) // 2, chunk_loop, 0)

    iter_body(0, NCH - 3)
    wait_gather(1)
    issue_gather(0)
    compute_chunk(1, NCH - 2)
    wait_gather(0)
    compute_chunk(0, NCH - 1)

    plsc.subcore_barrier()
    for r in range(VSL // C):
        rs = s * VSL + r * C
        pltpu.sync_copy(agg_sh.at[pl.ds(rs, C)], pfb.at[pl.ds(0, C)])
        pltpu.sync_copy(pfb.at[pl.ds(0, C)], agg_hbm.at[c, pl.ds(rs, C)])


def _combine_body(v_ref, ms_ref, a0_ref, a1_ref, out_ref):
    out_ref[...] = v_ref[...] + ms_ref[...] * (a0_ref[0] + a1_ref[0])


def _combine(vf, ms, agg):
    row = lambda i: (i, 0)
    return pl.pallas_call(
        _combine_body,
        grid=(RB,),
        in_specs=[
            pl.BlockSpec((RBS, D), row),
            pl.BlockSpec((RBS, 1), row),
            pl.BlockSpec((1, RBS, D), lambda i: (0, i, 0)),
            pl.BlockSpec((1, RBS, D), lambda i: (1, i, 0)),
        ],
        out_specs=pl.BlockSpec((RBS, D), row),
        out_shape=jax.ShapeDtypeStruct((N, D), jnp.float32),
    )(vf, ms, agg, agg)


def kernel(program_graph_feature, voxel_feature, cross_edge_program_index,
           cross_edge_voxel_index, W_dec1, b_dec1, W_dec2, b_dec2, W_v, b_v,
           W_p, b_p, theta):
    nkey = jax.random.key(42)
    k1, k2 = jax.random.split(nkey)
    u1 = jax.random.uniform(k1, (N, 2), jnp.float32, 1e-6, 1.0 - 1e-6)
    u2 = jax.random.uniform(k2, (E // 128, 128), jnp.float32, 1e-6, 1.0 - 1e-6)
    g2 = -jnp.log(-jnp.log(u2))

    av, ap, ms, mh = _dense(
        voxel_feature, program_graph_feature,
        W_v, b_v.reshape(1, D), W_p, b_p.reshape(1, D),
        W_dec1, b_dec1.reshape(1, D // 2), W_dec2, b_dec2.reshape(1, 2), u1)

    pad = lambda a: jnp.pad(a.reshape(NW, NCH, C), ((0, 0), (0, 0), (0, 128 - C)))
    vi3 = pad(cross_edge_voxel_index.astype(jnp.int32))
    pi3 = pad(cross_edge_program_index.astype(jnp.int32))
    pack_a = jnp.stack([vi3, pi3], axis=2).reshape(NW * NCH * 2 * 128)

    srows = _gsum(av, ap, pack_a)
    z2, ex2 = _att(srows, theta.reshape(1, D), g2.reshape(E, 1))

    zbits = pad(lax.bitcast_convert_type(z2, jnp.int32).reshape(NW, NCH, C))
    exbits = pad(lax.bitcast_convert_type(ex2, jnp.int32).reshape(NW, NCH, C))
    pack_d = jnp.stack([vi3, pi3, zbits, exbits],
                       axis=2).reshape(NW * NCH * 4 * 128)
    den_p, mx_p = _stats(pack_d)
    den_c, mx_c = _tables(den_p, mx_p)

    soft, hard, agg_p = _edge2(program_graph_feature, pack_d,
                               den_c.reshape(NVP), mx_c.reshape(NVP))

    nv = _combine(voxel_feature, ms, agg_p)
    return (mh, ms, hard[:, None], soft[:, None], nv)


# speedup vs baseline: 1.7688x; 1.7688x over previous
import functools

import jax
import jax.numpy as jnp
from jax import lax
from jax.experimental import pallas as pl
from jax.experimental.pallas import tpu as pltpu
from jax.experimental.pallas import tpu_sc as plsc

N = 10000
E = 320000
D = 128
NC, NS, L = 2, 16, 16
NW = NC * NS
EPW = E // NW
C = 80
NCH = EPW // C
GPC = C // L
NVP = 10240
VSL = NVP // NS
RB = 10
RBS = N // RB
EB = 160
EBS = E // EB

_mesh = plsc.VectorSubcoreMesh(
    core_axis_name="c", subcore_axis_name="s", num_cores=NC, num_subcores=NS)
_sc_params = pltpu.CompilerParams(needs_layout_passes=False)


def _dense_body(v_ref, p_ref, wv_ref, bv_ref, wp_ref, bp_ref, w1_ref, b1_ref,
                w2_ref, b2_ref, g1_ref, av_ref, ap_ref, ms_ref, mh_ref):
    v = v_ref[...]
    p = p_ref[...]
    dn = (((1,), (1,)), ((), ()))
    av_ref[...] = lax.dot_general(v, wv_ref[...], dn,
                                  preferred_element_type=jnp.float32) + bv_ref[...]
    ap_ref[...] = lax.dot_general(p, wp_ref[...], dn,
                                  preferred_element_type=jnp.float32) + bp_ref[...]
    h = lax.dot_general(v, w1_ref[...], dn,
                        preferred_element_type=jnp.float32) + b1_ref[...]
    logits = lax.dot_general(h, w2_ref[...], dn,
                             preferred_element_type=jnp.float32) + b2_ref[...]
    z = logits - jnp.log(-jnp.log(g1_ref[...]))
    z0 = z[:, 0:1]
    z1 = z[:, 1:2]
    m = jnp.maximum(z0, z1)
    e0 = jnp.exp(z0 - m)
    e1 = jnp.exp(z1 - m)
    ms_ref[...] = e0 / (e0 + e1)
    mh_ref[...] = (z0 >= z1).astype(jnp.float32)


def _dense(vf, pgf, wv, bv, wp, bp, w1, b1, w2, b2, g1):
    row = lambda i: (i, 0)
    whole = lambda i: (0, 0)
    return pl.pallas_call(
        _dense_body,
        grid=(RB,),
        in_specs=[
            pl.BlockSpec((RBS, D), row),
            pl.BlockSpec((RBS, D), row),
            pl.BlockSpec((D, D), whole),
            pl.BlockSpec((1, D), whole),
            pl.BlockSpec((D, D), whole),
            pl.BlockSpec((1, D), whole),
            pl.BlockSpec((D // 2, D), whole),
            pl.BlockSpec((1, D // 2), whole),
            pl.BlockSpec((2, D // 2), whole),
            pl.BlockSpec((1, 2), whole),
            pl.BlockSpec((RBS, 2), row),
        ],
        out_specs=[
            pl.BlockSpec((RBS, D), row),
            pl.BlockSpec((RBS, D), row),
            pl.BlockSpec((RBS, 1), row),
            pl.BlockSpec((RBS, 1), row),
        ],
        out_shape=[
            jax.ShapeDtypeStruct((N, D), jnp.float32),
            jax.ShapeDtypeStruct((N, D), jnp.float32),
            jax.ShapeDtypeStruct((N, 1), jnp.float32),
            jax.ShapeDtypeStruct((N, 1), jnp.float32),
        ],
    )(vf, pgf, wv, bv, wp, bp, w1, b1, w2, b2, g1)


@functools.partial(
    pl.kernel,
    out_type=jax.ShapeDtypeStruct((E, D), jnp.float32),
    mesh=_mesh,
    compiler_params=_sc_params,
    scratch_types=[
        pltpu.VMEM((2, 2 * 128), jnp.int32),
        pltpu.VMEM((2 * C, D), jnp.float32),
        pltpu.VMEM((2 * C, D), jnp.float32),
        pltpu.VMEM((2 * C, D), jnp.float32),
        pltpu.SemaphoreType.DMA((2,)),
        pltpu.SemaphoreType.DMA((2,)),
        pltpu.SemaphoreType.DMA((2,)),
        pltpu.SemaphoreType.DMA,
    ],
)
def _gsum(av_hbm, ap_hbm, pk_hbm, s_hbm,
          pk, avb, apb, sb, sem_av, sem_ap, sem_out, sem_idx):
    c = lax.axis_index("c")
    s = lax.axis_index("s")
    wid = c * NS + s
    pkb = wid * (NCH * 256)

    def issue_gathers(slot):
        pltpu.async_copy(av_hbm.at[pk.at[slot, pl.ds(0, C)]],
                         avb.at[pl.ds(slot * C, C)], sem_av.at[slot])
        pltpu.async_copy(ap_hbm.at[pk.at[slot, pl.ds(128, C)]],
                         apb.at[pl.ds(slot * C, C)], sem_ap.at[slot])

    def wait_gathers(slot):
        pltpu.make_async_copy(av_hbm.at[pk.at[slot, pl.ds(0, C)]],
                              avb.at[pl.ds(slot * C, C)],
                              sem_av.at[slot]).wait()
        pltpu.make_async_copy(ap_hbm.at[pk.at[slot, pl.ds(128, C)]],
                              apb.at[pl.ds(slot * C, C)],
                              sem_ap.at[slot]).wait()

    def compute_chunk(b, ch, wait_prev_out):
        base = wid * EPW + ch * C

        def row_block(g, _):
            off = g * L
            for e in range(L):
                row = b * C + off + e
                for j in range(D // L):
                    sl = pl.ds(j * L, L)
                    sb[row, sl] = avb[row, sl] + apb[row, sl]
            return 0
        lax.fori_loop(0, GPC, row_block, 0)
        if wait_prev_out:
            pltpu.make_async_copy(
                sb.at[pl.ds(b * C, C)],
                s_hbm.at[pl.ds(wid * EPW + (ch - 2) * C, C)],
                sem_out.at[b]).wait()
        pltpu.async_copy(sb.at[pl.ds(b * C, C)], s_hbm.at[pl.ds(base, C)],
                         sem_out.at[b])

    def iter_body(b, ch, wait_prev_out):
        wait_gathers(b)
        issue_gathers(1 - b)
        d = pltpu.async_copy(pk_hbm.at[pl.ds(pkb + (ch + 2) * 256, 256)],
                             pk.at[b], sem_idx)
        compute_chunk(b, ch, wait_prev_out)
        d.wait()

    pltpu.sync_copy(pk_hbm.at[pl.ds(pkb, 256)], pk.at[0])
    issue_gathers(0)
    pltpu.sync_copy(pk_hbm.at[pl.ds(pkb + 256, 256)], pk.at[1])

    iter_body(0, 0, False)
    iter_body(1, 1, False)

    def chunk_loop(k, _):
        iter_body(0, 2 * k + 2, True)
        iter_body(1, 2 * k + 3, True)
        return 0
    lax.fori_loop(0, (NCH - 5) // 2, chunk_loop, 0)

    iter_body(0, NCH - 3, True)
    wait_gathers(1)
    issue_gathers(0)
    compute_chunk(1, NCH - 2, True)
    wait_gathers(0)
    compute_chunk(0, NCH - 1, True)
    pltpu.make_async_copy(sb.at[pl.ds(C, C)],
                          s_hbm.at[pl.ds(wid * EPW + (NCH - 2) * C, C)],
                          sem_out.at[1]).wait()
    pltpu.make_async_copy(sb.at[pl.ds(0, C)],
                          s_hbm.at[pl.ds(wid * EPW + (NCH - 1) * C, C)],
                          sem_out.at[0]).wait()


AB = 500
ABS = E // AB
AG = ABS // 128


def _att_body(s_ref, th_ref, u_ref, z_ref, ex_ref):
    t = jnp.tanh(s_ref[...])
    g2 = -jnp.log(-jnp.log(u_ref[0]))
    dn = (((1,), (1,)), ((), ()))
    rows = []
    for j in range(AG):
        tj = t[j * 128:(j + 1) * 128, :]
        rows.append(lax.dot_general(th_ref[...], tj, dn,
                                    precision=lax.Precision.HIGHEST,
                                    preferred_element_type=jnp.float32))
    z = jnp.concatenate(rows, axis=0) + g2
    z_ref[0] = z
    ex_ref[0] = jnp.exp(z)


def _att(s, theta, u3):
    return pl.pallas_call(
        _att_body,
        grid=(AB,),
        in_specs=[
            pl.BlockSpec((ABS, D), lambda i: (i, 0)),
            pl.BlockSpec((1, D), lambda i: (0, 0)),
            pl.BlockSpec((1, AG, 128), lambda i: (i, 0, 0)),
        ],
        out_specs=[
            pl.BlockSpec((1, AG, 128), lambda i: (i, 0, 0)),
            pl.BlockSpec((1, AG, 128), lambda i: (i, 0, 0)),
        ],
        out_shape=[
            jax.ShapeDtypeStruct((AB, AG, 128), jnp.float32),
            jax.ShapeDtypeStruct((AB, AG, 128), jnp.float32),
        ],
    )(s, theta, u3)


@functools.partial(
    pl.kernel,
    out_type=[
        jax.ShapeDtypeStruct((NC, NVP), jnp.float32),
        jax.ShapeDtypeStruct((NC, NVP), jnp.float32),
    ],
    mesh=_mesh,
    compiler_params=_sc_params,
    scratch_types=[
        pltpu.VMEM((2, 4 * 128), jnp.int32),
        pltpu.VMEM((C,), jnp.int32),
        pltpu.VMEM((C,), jnp.float32),
        pltpu.VMEM((C,), jnp.float32),
        pltpu.VMEM((NVP,), jnp.float32),
        pltpu.VMEM((NS, VSL), jnp.float32),
        pltpu.VMEM((VSL,), jnp.float32),
        pltpu.VMEM_SHARED((NVP,), jnp.float32),
        pltpu.VMEM_SHARED((NS, NVP), jnp.float32),
        pltpu.SemaphoreType.DMA,
    ],
)
def _stats(pk_hbm, den_hbm, mx_hbm,
           pk, vi_s, zc_v, exc_v, mx_tbl, red_v, slice_v, den_sh, mx_sh,
           sem_idx):
    c = lax.axis_index("c")
    s = lax.axis_index("s")
    wid = c * NS + s
    pkb = wid * (NCH * 512)

    neg = jnp.full((L,), -1e30, jnp.float32)

    def fill_mx(i, _):
        mx_tbl[pl.ds(i * L, L)] = neg
        return 0
    lax.fori_loop(0, NVP // L, fill_mx, 0)

    zv = jnp.zeros((L,), jnp.float32)

    def fill_z(i, _):
        slice_v[pl.ds(i * L, L)] = zv
        return 0
    lax.fori_loop(0, VSL // L, fill_z, 0)
    pltpu.sync_copy(slice_v, den_sh.at[pl.ds(s * VSL, VSL)])
    plsc.subcore_barrier()

    def compute_chunk(b, ch):
        for j in range(C // L):
            sl = pl.ds(j * L, L)
            vi_s[sl] = pk[b, pl.ds(j * L, L)]
            zc_v[sl] = plsc.bitcast(pk[b, pl.ds(256 + j * L, L)], jnp.float32)
            exc_v[sl] = plsc.bitcast(pk[b, pl.ds(384 + j * L, L)], jnp.float32)

        def group_body(g, _):
            off = g * L
            z16 = zc_v[pl.ds(off, L)]
            vi16 = vi_s[pl.ds(off, L)]

            def mx_step(pending):
                cur = plsc.load_gather(mx_tbl, [vi16])
                need = jnp.logical_and(pending, z16 > cur)
                plsc.store_scatter(mx_tbl, [vi16], z16, mask=need)
                cur2 = plsc.load_gather(mx_tbl, [vi16])
                return jnp.logical_and(need, z16 > cur2)
            lax.while_loop(lambda p: jnp.any(p), mx_step,
                           jnp.ones((L,), jnp.bool_))
            return 0
        lax.fori_loop(0, GPC, group_body, 0)
        pltpu.sync_copy(exc_v, den_sh.at[vi_s], add=True)

    def iter_body(b, ch):
        d = pltpu.async_copy(pk_hbm.at[pl.ds(pkb + (ch + 1) * 512, 512)],
                             pk.at[1 - b], sem_idx)
        compute_chunk(b, ch)
        d.wait()

    pltpu.sync_copy(pk_hbm.at[pl.ds(pkb, 512)], pk.at[0])

    def chunk_loop(k, _):
        iter_body(0, 2 * k)
        iter_body(1, 2 * k + 1)
        return 0
    lax.fori_loop(0, (NCH - 1) // 2, chunk_loop, 0)
    compute_chunk(0, NCH - 1)

    plsc.subcore_barrier()
    pltpu.sync_copy(mx_tbl, mx_sh.at[s])
    plsc.subcore_barrier()
    for j in range(NS):
        pltpu.sync_copy(mx_sh.at[j, pl.ds(s * VSL, VSL)], red_v.at[j])

    def red_max(k, _):
        m = red_v[0, pl.ds(k * L, L)]
        for j in range(1, NS):
            m = jnp.maximum(m, red_v[j, pl.ds(k * L, L)])
        slice_v[pl.ds(k * L, L)] = m
        return 0
    lax.fori_loop(0, VSL // L, red_max, 0)
    pltpu.sync_copy(slice_v, mx_hbm.at[c, pl.ds(s * VSL, VSL)])

    pltpu.sync_copy(den_sh.at[pl.ds(s * VSL, VSL)], slice_v)
    pltpu.sync_copy(slice_v, den_hbm.at[c, pl.ds(s * VSL, VSL)])


def _tables_body(denp_ref, mxp_ref, den_ref, mx_ref):
    den_ref[...] = denp_ref[0:1, :] + denp_ref[1:2, :]
    mx_ref[...] = jnp.maximum(mxp_ref[0:1, :], mxp_ref[1:2, :])


def _tables(den_p, mx_p):
    whole = lambda: (0, 0)
    return pl.pallas_call(
        _tables_body,
        grid=(),
        in_specs=[pl.BlockSpec((NC, NVP), whole),
                  pl.BlockSpec((NC, NVP), whole)],
        out_specs=[pl.BlockSpec((1, NVP), whole),
                   pl.BlockSpec((1, NVP), whole)],
        out_shape=[jax.ShapeDtypeStruct((1, NVP), jnp.float32),
                   jax.ShapeDtypeStruct((1, NVP), jnp.float32)],
    )(den_p, mx_p)


@functools.partial(
    pl.kernel,
    out_type=[
        jax.ShapeDtypeStruct((E,), jnp.float32),
        jax.ShapeDtypeStruct((E,), jnp.float32),
        jax.ShapeDtypeStruct((NC, NVP, D), jnp.float32),
    ],
    mesh=_mesh,
    compiler_params=_sc_params,
    scratch_types=[
        pltpu.VMEM((2, 4 * 128), jnp.int32),
        pltpu.VMEM((2 * C, D), jnp.float32),
        pltpu.VMEM((C,), jnp.int32),
        pltpu.VMEM((C,), jnp.float32),
        pltpu.VMEM((C,), jnp.float32),
        pltpu.VMEM((C,), jnp.float32),
        pltpu.VMEM((C,), jnp.float32),
        pltpu.VMEM((NVP,), jnp.float32),
        pltpu.VMEM((NVP,), jnp.float32),
        pltpu.VMEM_SHARED((NVP, D), jnp.float32),
        pltpu.SemaphoreType.DMA((2,)),
        pltpu.SemaphoreType.DMA,
    ],
)
def _edge2(pgf_hbm, pk_hbm, den_hbm, mx_hbm,
           soft_hbm, hard_hbm, agg_hbm,
           pk, pfb, vi_s, zc_v, exc_v, softb, hardb, mx_tbl, den_tbl,
           agg_sh, sem_pf, sem_idx):
    c = lax.axis_index("c")
    s = lax.axis_index("s")
    wid = c * NS + s
    pkb = wid * (NCH * 512)

    pltpu.sync_copy(mx_hbm, mx_tbl)
    pltpu.sync_copy(den_hbm, den_tbl)

    zv = jnp.zeros((L,), jnp.float32)
    for i in range(L):
        for j in range(D // L):
            pfb[i, pl.ds(j * L, L)] = zv
    for r in range(VSL // L):
        pltpu.sync_copy(pfb.at[pl.ds(0, L)],
                        agg_sh.at[pl.ds(s * VSL + r * L, L)])
    plsc.subcore_barrier()

    def issue_gather(slot):
        pltpu.async_copy(pgf_hbm.at[pk.at[slot, pl.ds(128, C)]],
                         pfb.at[pl.ds(slot * C, C)], sem_pf.at[slot])

    def wait_gather(slot):
        pltpu.make_async_copy(pgf_hbm.at[pk.at[slot, pl.ds(128, C)]],
                              pfb.at[pl.ds(slot * C, C)],
                              sem_pf.at[slot]).wait()

    def compute_chunk(b, ch):
        base = wid * EPW + ch * C
        for j in range(C // L):
            sl = pl.ds(j * L, L)
            vi_s[sl] = pk[b, pl.ds(j * L, L)]
            zc_v[sl] = plsc.bitcast(pk[b, pl.ds(256 + j * L, L)], jnp.float32)
            exc_v[sl] = plsc.bitcast(pk[b, pl.ds(384 + j * L, L)], jnp.float32)

        def group_body(g, _):
            off = g * L
            z16 = zc_v[pl.ds(off, L)]
            vi16 = vi_s[pl.ds(off, L)]
            d16 = plsc.load_gather(den_tbl, [vi16])
            m16 = plsc.load_gather(mx_tbl, [vi16])
            soft16 = exc_v[pl.ds(off, L)] / d16
            softb[pl.ds(off, L)] = soft16
            hardb[pl.ds(off, L)] = jnp.where(z16 >= m16, 1.0, 0.0)
            for e in range(L):
                row = off + e
                sc = soft16[e]
                for j in range(D // L):
                    sl = pl.ds(j * L, L)
                    pfb[b * C + row, sl] = pfb[b * C + row, sl] * sc
            return 0
        lax.fori_loop(0, GPC, group_body, 0)

        pltpu.sync_copy(pfb.at[pl.ds(b * C, C)], agg_sh.at[vi_s], add=True)
        pltpu.sync_copy(softb, soft_hbm.at[pl.ds(base, C)])
        pltpu.sync_copy(hardb, hard_hbm.at[pl.ds(base, C)])

    pltpu.sync_copy(pk_hbm.at[pl.ds(pkb, 512)], pk.at[0])
    issue_gather(0)
    pltpu.sync_copy(pk_hbm.at[pl.ds(pkb + 512, 512)], pk.at[1])

    def iter_body(b, ch):
        wait_gather(b)
        issue_gather(1 - b)
        d = pltpu.async_copy(pk_hbm.at[pl.ds(pkb + (ch + 2) * 512, 512)],
                             pk.at[b], sem_idx)
        compute_chunk(b, ch)
        d.wait()

    def chunk_loop(k, _):
        iter_body(0, 2 * k)
        iter_body(1, 2 * k + 1)
        return 0
    lax.fori_loop(0, (NCH - 3) // 2, chunk_loop, 0)

    iter_body(0, NCH - 3)
    wait_gather(1)
    issue_gather(0)
    compute_chunk(1, NCH - 2)
    wait_gather(0)
    compute_chunk(0, NCH - 1)

    plsc.subcore_barrier()
    for r in range(VSL // C):
        rs = s * VSL + r * C
        pltpu.sync_copy(agg_sh.at[pl.ds(rs, C)], pfb.at[pl.ds(0, C)])
        pltpu.sync_copy(pfb.at[pl.ds(0, C)], agg_hbm.at[c, pl.ds(rs, C)])


def _combine_body(v_ref, ms_ref, a0_ref, a1_ref, out_ref):
    out_ref[...] = v_ref[...] + ms_ref[...] * (a0_ref[0] + a1_ref[0])


def _combine(vf, ms, agg):
    row = lambda i: (i, 0)
    return pl.pallas_call(
        _combine_body,
        grid=(RB,),
        in_specs=[
            pl.BlockSpec((RBS, D), row),
            pl.BlockSpec((RBS, 1), row),
            pl.BlockSpec((1, RBS, D), lambda i: (0, i, 0)),
            pl.BlockSpec((1, RBS, D), lambda i: (1, i, 0)),
        ],
        out_specs=pl.BlockSpec((RBS, D), row),
        out_shape=jax.ShapeDtypeStruct((N, D), jnp.float32),
    )(vf, ms, agg, agg)


def kernel(program_graph_feature, voxel_feature, cross_edge_program_index,
           cross_edge_voxel_index, W_dec1, b_dec1, W_dec2, b_dec2, W_v, b_v,
           W_p, b_p, theta):
    nkey = jax.random.key(42)
    k1, k2 = jax.random.split(nkey)
    u1 = jax.random.uniform(k1, (N, 2), jnp.float32, 1e-6, 1.0 - 1e-6)
    u2 = jax.random.uniform(k2, (E // 128, 128), jnp.float32, 1e-6, 1.0 - 1e-6)

    av, ap, ms, mh = _dense(
        voxel_feature, program_graph_feature,
        W_v, b_v.reshape(1, D), W_p, b_p.reshape(1, D),
        W_dec1, b_dec1.reshape(1, D // 2), W_dec2, b_dec2.reshape(1, 2), u1)

    pad = lambda a: jnp.pad(a.reshape(NW, NCH, C), ((0, 0), (0, 0), (0, 128 - C)))
    vi3 = pad(cross_edge_voxel_index.astype(jnp.int32))
    pi3 = pad(cross_edge_program_index.astype(jnp.int32))
    pack_a = jnp.stack([vi3, pi3], axis=2).reshape(NW * NCH * 2 * 128)

    srows = _gsum(av, ap, pack_a)
    z2, ex2 = _att(srows, theta.reshape(1, D), u2.reshape(AB, AG, 128))

    zbits = pad(lax.bitcast_convert_type(z2, jnp.int32).reshape(NW, NCH, C))
    exbits = pad(lax.bitcast_convert_type(ex2, jnp.int32).reshape(NW, NCH, C))
    pack_d = jnp.stack([vi3, pi3, zbits, exbits],
                       axis=2).reshape(NW * NCH * 4 * 128)
    den_p, mx_p = _stats(pack_d)
    den_c, mx_c = _tables(den_p, mx_p)

    soft, hard, agg_p = _edge2(program_graph_feature, pack_d,
                               den_c.reshape(NVP), mx_c.reshape(NVP))

    nv = _combine(voxel_feature, ms, agg_p)
    return (mh, ms, hard[:, None], soft[:, None], nv)
